# baseline (speedup 1.0000x reference)
import jax
import jax.numpy as jnp
from jax.experimental import pallas as pl

_E_BLOCK = 2000


def _rbf_body(d_ref, c_ref, o_ref):
    c = c_ref[...]
    gap = c[0, 1] - c[0, 0]
    coef = -1.0 / gap
    d = d_ref[...]
    r = d - c
    o_ref[...] = jnp.exp(coef * (r * r))


def kernel(distance, centers):
    E = distance.shape[0]
    N = centers.shape[0]
    c2 = centers.reshape(1, N)
    grid = (E // _E_BLOCK,)
    return pl.pallas_call(
        _rbf_body,
        grid=grid,
        in_specs=[
            pl.BlockSpec((_E_BLOCK, 1), lambda i: (i, 0)),
            pl.BlockSpec((1, N), lambda i: (0, 0)),
        ],
        out_specs=pl.BlockSpec((_E_BLOCK, N), lambda i: (i, 0)),
        out_shape=jax.ShapeDtypeStruct((E, N), jnp.float32),
    )(distance, c2)
